# baseline (speedup 1.0000x reference)
import functools

import jax
import jax.numpy as jnp
from jax import lax
from jax.experimental import pallas as pl
from jax.experimental.pallas import tpu as pltpu
from jax.experimental.pallas import tpu_sc as plsc

VOCAB = 100000
DIM = 128
B = 16384
K = 20

_INFO = plsc.get_sparse_core_info()
NC = _INFO.num_cores
NS = _INFO.num_subcores
LANES = _INFO.num_lanes
NW = NC * NS
BPW = B // NW
SB = 8
CR = SB * K
NCHUNK = BPW // SB
DCH = DIM // LANES


def _make_sc_kernel():
    mesh = plsc.VectorSubcoreMesh(core_axis_name="c", subcore_axis_name="s")

    @functools.partial(
        pl.kernel,
        mesh=mesh,
        compiler_params=pltpu.CompilerParams(needs_layout_passes=False,
                                             use_tc_tiling_on_sc=False),
        out_type=jax.ShapeDtypeStruct((B * K,), jnp.float32),
        scratch_types=[
            pltpu.VMEM((BPW,), jnp.int32),
            pltpu.VMEM((BPW * K,), jnp.int32),
            pltpu.VMEM((SB, DIM // 2), jnp.int32),
            pltpu.VMEM((SB, DIM // 2), jnp.int32),
            pltpu.VMEM((CR, DIM // 2), jnp.int32),
            pltpu.VMEM((CR, DIM // 2), jnp.int32),
            pltpu.VMEM((BPW * K,), jnp.float32),
            pltpu.SemaphoreType.DMA,
            pltpu.SemaphoreType.DMA,
        ],
    )
    def sc_kernel(tgt_hbm, ctx_hbm, in_hbm, outemb_hbm, scores_hbm,
                  tgt_idx, ctx_idx, tgt_a, tgt_b, ctx_a, ctx_b, out_v,
                  sem_a, sem_b):
        wid = lax.axis_index("s") * NC + lax.axis_index("c")
        base_b = wid * BPW
        base_f = wid * (BPW * K)
        pltpu.sync_copy(tgt_hbm.at[pl.ds(base_b, BPW)], tgt_idx)
        pltpu.sync_copy(ctx_hbm.at[pl.ds(base_f, BPW * K)], ctx_idx)

        def fire(s, tbuf, cbuf, sem):
            pltpu.async_copy(in_hbm.at[tgt_idx.at[pl.ds(s * SB, SB)]],
                             tbuf, sem)
            pltpu.async_copy(
                outemb_hbm.at[ctx_idx.at[pl.ds(s * CR, CR // 2)]],
                cbuf.at[pl.ds(0, CR // 2)], sem)
            pltpu.async_copy(
                outemb_hbm.at[ctx_idx.at[pl.ds(s * CR + CR // 2, CR // 2)]],
                cbuf.at[pl.ds(CR // 2, CR // 2)], sem)

        def drain(tbuf, cbuf, sem):
            pltpu.make_async_copy(in_hbm.at[pl.ds(0, SB)], tbuf, sem).wait()
            pltpu.make_async_copy(outemb_hbm.at[pl.ds(0, CR // 2)],
                                  cbuf.at[pl.ds(0, CR // 2)], sem).wait()
            pltpu.make_async_copy(outemb_hbm.at[pl.ds(0, CR // 2)],
                                  cbuf.at[pl.ds(CR // 2, CR // 2)], sem).wait()

        def unpack_row(buf, row):
            parts = []
            for h in range(DIM // 32):
                w = buf[row, pl.ds(h * LANES, LANES)]
                ab = plsc.bitcast(w, jnp.bfloat16)
                parts.extend(plsc.unpack(ab, format=plsc.PackFormat.INTERLEAVED))
            return parts

        def compute(s, tbuf, cbuf):
            lane = lax.iota(jnp.int32, LANES)
            tcache = {}
            for g in range(CR // LANES):
                group = jnp.zeros((LANES,), jnp.float32)
                for m in range(LANES):
                    j = g * LANES + m
                    bb = j // K
                    if bb not in tcache:
                        tcache[bb] = unpack_row(tbuf, bb)
                    t = tcache[bb]
                    c = unpack_row(cbuf, j)
                    acc = t[0] * c[0]
                    for d in range(1, DCH):
                        acc = acc + t[d] * c[d]
                    group = jnp.where(lane == m, jnp.sum(acc), group)
                out_v[pl.ds(s * CR + g * LANES, LANES)] = group

        fire(0, tgt_a, ctx_a, sem_a)

        def pair(p, carry):
            s0 = 2 * p
            fire(s0 + 1, tgt_b, ctx_b, sem_b)
            drain(tgt_a, ctx_a, sem_a)
            compute(s0, tgt_a, ctx_a)

            @pl.when(p < NCHUNK // 2 - 1)
            def _():
                fire(s0 + 2, tgt_a, ctx_a, sem_a)

            drain(tgt_b, ctx_b, sem_b)
            compute(s0 + 1, tgt_b, ctx_b)
            return carry

        lax.fori_loop(0, NCHUNK // 2, pair, 0)
        pltpu.sync_copy(out_v, scores_hbm.at[pl.ds(base_f, BPW * K)])

    return sc_kernel


_SC_KERNEL = _make_sc_kernel()


def kernel(target, context, in_embed, out_embed):
    tgt = target.astype(jnp.int32)
    ctx = context.astype(jnp.int32).reshape(-1)
    def _pack_bf16(table):
        bf = table.astype(jnp.bfloat16)
        return lax.bitcast_convert_type(
            bf.reshape(table.shape[0], table.shape[1] // 2, 2), jnp.int32)

    scores = _SC_KERNEL(tgt, ctx, _pack_bf16(in_embed), _pack_bf16(out_embed))
    return scores.reshape(context.shape[0], context.shape[1])


# speedup vs baseline: 5.0490x; 5.0490x over previous
import functools

import jax
import jax.numpy as jnp
from jax import lax
from jax.experimental import pallas as pl
from jax.experimental.pallas import tpu as pltpu
from jax.experimental.pallas import tpu_sc as plsc

VOCAB = 100000
DIM = 128
B = 16384
K = 20

_INFO = plsc.get_sparse_core_info()
NC = _INFO.num_cores
NS = _INFO.num_subcores
LANES = _INFO.num_lanes
NW = NC * NS
BPW = B // NW
SB = 8
CR = SB * K
NCHUNK = BPW // SB
DCH = DIM // LANES


def _make_sc_kernel():
    mesh = plsc.VectorSubcoreMesh(core_axis_name="c", subcore_axis_name="s")

    @functools.partial(
        pl.kernel,
        mesh=mesh,
        compiler_params=pltpu.CompilerParams(needs_layout_passes=False),
        out_type=jax.ShapeDtypeStruct((B * K,), jnp.float32),
        scratch_types=[
            pltpu.VMEM((BPW,), jnp.int32),
            pltpu.VMEM((BPW * K,), jnp.int32),
            pltpu.VMEM((SB, DIM), jnp.float32),
            pltpu.VMEM((SB, DIM), jnp.float32),
            pltpu.VMEM((CR, DIM), jnp.float32),
            pltpu.VMEM((CR, DIM), jnp.float32),
            pltpu.VMEM((BPW * K,), jnp.float32),
            pltpu.SemaphoreType.DMA,
            pltpu.SemaphoreType.DMA,
        ],
    )
    def sc_kernel(tgt_hbm, ctx_hbm, in_hbm, outemb_hbm, scores_hbm,
                  tgt_idx, ctx_idx, tgt_a, tgt_b, ctx_a, ctx_b, out_v,
                  sem_a, sem_b):
        wid = lax.axis_index("s") * NC + lax.axis_index("c")
        base_b = wid * BPW
        base_f = wid * (BPW * K)
        pltpu.sync_copy(tgt_hbm.at[pl.ds(base_b, BPW)], tgt_idx)
        pltpu.sync_copy(ctx_hbm.at[pl.ds(base_f, BPW * K)], ctx_idx)

        NSTREAM = 4
        CSTR = CR // NSTREAM

        def fire(s, tbuf, cbuf, sem):
            pltpu.async_copy(in_hbm.at[tgt_idx.at[pl.ds(s * SB, SB)]],
                             tbuf, sem)
            for q in range(NSTREAM):
                pltpu.async_copy(
                    outemb_hbm.at[ctx_idx.at[pl.ds(s * CR + q * CSTR, CSTR)]],
                    cbuf.at[pl.ds(q * CSTR, CSTR)], sem)

        def drain(tbuf, cbuf, sem):
            pltpu.make_async_copy(in_hbm.at[pl.ds(0, SB)], tbuf, sem).wait()
            for q in range(NSTREAM):
                pltpu.make_async_copy(outemb_hbm.at[pl.ds(0, CSTR)],
                                      cbuf.at[pl.ds(q * CSTR, CSTR)],
                                      sem).wait()

        def compute(s, tbuf, cbuf):
            lane = lax.iota(jnp.int32, LANES)
            tcache = {}
            for g in range(CR // LANES):
                group = jnp.zeros((LANES,), jnp.float32)
                for m in range(LANES):
                    j = g * LANES + m
                    bb = j // K
                    if bb not in tcache:
                        tcache[bb] = [tbuf[bb, pl.ds(d * LANES, LANES)]
                                      for d in range(DCH)]
                    t = tcache[bb]
                    acc = t[0] * cbuf[j, pl.ds(0, LANES)]
                    for d in range(1, DCH):
                        acc = acc + t[d] * cbuf[j, pl.ds(d * LANES, LANES)]
                    group = jnp.where(lane == m, jnp.sum(acc), group)
                out_v[pl.ds(s * CR + g * LANES, LANES)] = group

        fire(0, tgt_a, ctx_a, sem_a)

        def pair(p, carry):
            s0 = 2 * p
            fire(s0 + 1, tgt_b, ctx_b, sem_b)
            drain(tgt_a, ctx_a, sem_a)
            compute(s0, tgt_a, ctx_a)

            @pl.when(p < NCHUNK // 2 - 1)
            def _():
                fire(s0 + 2, tgt_a, ctx_a, sem_a)

            drain(tgt_b, ctx_b, sem_b)
            compute(s0 + 1, tgt_b, ctx_b)
            return carry

        lax.fori_loop(0, NCHUNK // 2, pair, 0)
        pltpu.sync_copy(out_v, scores_hbm.at[pl.ds(base_f, BPW * K)])

    return sc_kernel


_SC_KERNEL = _make_sc_kernel()


def kernel(target, context, in_embed, out_embed):
    tgt = target.astype(jnp.int32)
    ctx = context.astype(jnp.int32).reshape(-1)
    scores = _SC_KERNEL(tgt, ctx, in_embed, out_embed)
    return scores.reshape(context.shape[0], context.shape[1])


# speedup vs baseline: 5.3448x; 1.0586x over previous
import functools

import jax
import jax.numpy as jnp
from jax import lax
from jax.experimental import pallas as pl
from jax.experimental.pallas import tpu as pltpu
from jax.experimental.pallas import tpu_sc as plsc

VOCAB = 100000
DIM = 128
B = 16384
K = 20

_INFO = plsc.get_sparse_core_info()
NC = _INFO.num_cores
NS = _INFO.num_subcores
LANES = _INFO.num_lanes
NW = NC * NS
BPW = B // NW
SB = 8
CR = SB * K
NCHUNK = BPW // SB
DCH = DIM // LANES

_DIV20_MAGIC = 52429
_DIV20_SHIFT = 20


def _make_sc_kernel():
    mesh = plsc.VectorSubcoreMesh(core_axis_name="c", subcore_axis_name="s")

    @functools.partial(
        pl.kernel,
        mesh=mesh,
        compiler_params=pltpu.CompilerParams(needs_layout_passes=False),
        out_type=jax.ShapeDtypeStruct((B, K), jnp.float32),
        scratch_types=[
            pltpu.VMEM((BPW,), jnp.int32),
            pltpu.VMEM((BPW * K,), jnp.int32),
            pltpu.VMEM((SB, DIM), jnp.float32),
            pltpu.VMEM((SB, DIM), jnp.float32),
            pltpu.VMEM((CR, DIM), jnp.float32),
            pltpu.VMEM((CR, DIM), jnp.float32),
            pltpu.VMEM((BPW, K), jnp.float32),
            pltpu.SemaphoreType.DMA,
            pltpu.SemaphoreType.DMA,
        ],
    )
    def sc_kernel(tgt_hbm, ctx_hbm, in_hbm, outemb_hbm, scores_hbm,
                  tgt_idx, ctx_idx, tgt_a, tgt_b, ctx_a, ctx_b, out_v,
                  sem_a, sem_b):
        wid = lax.axis_index("s") * NC + lax.axis_index("c")
        base_b = wid * BPW
        base_f = wid * (BPW * K)
        pltpu.sync_copy(tgt_hbm.at[pl.ds(base_b, BPW)], tgt_idx)
        pltpu.sync_copy(ctx_hbm.at[pl.ds(base_f, BPW * K)], ctx_idx)

        def fire(s, tbuf, cbuf, sem):
            pltpu.async_copy(in_hbm.at[tgt_idx.at[pl.ds(s * SB, SB)]],
                             tbuf, sem)
            pltpu.async_copy(
                outemb_hbm.at[ctx_idx.at[pl.ds(s * CR, CR // 2)]],
                cbuf.at[pl.ds(0, CR // 2)], sem)
            pltpu.async_copy(
                outemb_hbm.at[ctx_idx.at[pl.ds(s * CR + CR // 2, CR // 2)]],
                cbuf.at[pl.ds(CR // 2, CR // 2)], sem)

        def drain(tbuf, cbuf, sem):
            pltpu.make_async_copy(in_hbm.at[pl.ds(0, SB)], tbuf, sem).wait()
            for q in range(2):
                pltpu.make_async_copy(outemb_hbm.at[pl.ds(0, CR // 2)],
                                      cbuf.at[pl.ds(q * (CR // 2), CR // 2)],
                                      sem).wait()

        lane = lax.iota(jnp.int32, LANES)

        def compute(s, tbuf, cbuf):
            tcache = {}
            for g in range(CR // LANES):
                group = jnp.zeros((LANES,), jnp.float32)
                for m in range(LANES):
                    j = g * LANES + m
                    bb = j // K
                    k = j % K
                    if bb not in tcache:
                        tcache[bb] = [tbuf[bb, pl.ds(d * LANES, LANES)]
                                      for d in range(DCH)]
                    t = tcache[bb]
                    acc = t[0] * cbuf[j, pl.ds(0, LANES)]
                    for d in range(1, DCH):
                        acc = acc + t[d] * cbuf[j, pl.ds(d * LANES, LANES)]
                    group = jnp.where(lane == m, jnp.sum(acc), group)
                flat = s * CR + g * LANES + lane
                row = lax.shift_right_logical(flat * _DIV20_MAGIC,
                                              _DIV20_SHIFT)
                col = flat - row * K
                plsc.store_scatter(out_v, [row, col], group)

        fire(0, tgt_a, ctx_a, sem_a)

        def pair(p, carry):
            s0 = 2 * p
            fire(s0 + 1, tgt_b, ctx_b, sem_b)
            drain(tgt_a, ctx_a, sem_a)
            compute(s0, tgt_a, ctx_a)

            @pl.when(p < NCHUNK // 2 - 1)
            def _():
                fire(s0 + 2, tgt_a, ctx_a, sem_a)

            drain(tgt_b, ctx_b, sem_b)
            compute(s0 + 1, tgt_b, ctx_b)
            return carry

        lax.fori_loop(0, NCHUNK // 2, pair, 0)
        pltpu.sync_copy(out_v, scores_hbm.at[pl.ds(base_b, BPW), :])

    return sc_kernel


_SC_KERNEL = _make_sc_kernel()


def kernel(target, context, in_embed, out_embed):
    tgt = target.astype(jnp.int32)
    ctx = context.astype(jnp.int32).reshape(-1)
    return _SC_KERNEL(tgt, ctx, in_embed, out_embed)
